# baseline (speedup 1.0000x reference)
import functools

import jax
import jax.numpy as jnp
from jax import lax
from jax.experimental import pallas as pl
from jax.experimental.pallas import tpu as pltpu
from jax.experimental.pallas import tpu_sc as plsc

_NUM_CORES = 2
_NUM_SUBCORES = 16
_NUM_WORKERS = _NUM_CORES * _NUM_SUBCORES
_CHUNK = 128
_K = 4
_LANES = 16


@functools.lru_cache(maxsize=None)
def _make_gather(V, D, S, B0):
    b_per_w = B0 // _NUM_WORKERS
    k_per_s = b_per_w // _CHUNK
    n_tiles = S * k_per_s
    n_groups = n_tiles // _K
    assert n_groups % 2 == 0 and n_groups >= 4
    mesh = plsc.VectorSubcoreMesh(core_axis_name="c", subcore_axis_name="s")

    @functools.partial(
        pl.kernel,
        mesh=mesh,
        out_type=jax.ShapeDtypeStruct((S, D, B0), jnp.float32),
        scratch_types=[
            pltpu.VMEM((S, b_per_w), jnp.int32),
            pltpu.VMEM((_K, _CHUNK, D), jnp.float32),
            pltpu.VMEM((_K, _CHUNK, D), jnp.float32),
            pltpu.VMEM((_K, D, _CHUNK), jnp.float32),
            pltpu.SemaphoreType.DMA,
            pltpu.SemaphoreType.DMA,
            pltpu.SemaphoreType.DMA,
        ],
        compiler_params=pltpu.CompilerParams(
            use_tc_tiling_on_sc=False, needs_layout_passes=False),
    )
    def gather_kernel(idx_hbm, table_hbm, out_hbm, idx_v,
                      rows_a, rows_b, tsp_v, gs_a, gs_b, os_t):
        wid = lax.axis_index("s") * _NUM_CORES + lax.axis_index("c")
        col0 = wid * b_per_w
        pltpu.sync_copy(idx_hbm.at[:, pl.ds(col0, b_per_w)], idx_v)

        lane = lax.iota(jnp.int32, _LANES)
        row_vecs = [lane + m * _LANES for m in range(_CHUNK // _LANES)]

        def tile_sb(t):
            return t // k_per_s, (t % k_per_s) * _CHUNK

        def fire_gathers(g, rows, sem):
            for b in range(_K):
                s, boff = tile_sb(g * _K + b)
                pltpu.async_copy(
                    table_hbm.at[idx_v.at[s, pl.ds(boff, _CHUNK)]],
                    rows.at[b], sem)

        def drain_gathers(g, rows, sem):
            for b in range(_K):
                s, boff = tile_sb(g * _K + b)
                pltpu.make_async_copy(
                    table_hbm.at[idx_v.at[s, pl.ds(boff, _CHUNK)]],
                    rows.at[b], sem).wait()

        def transpose_group(rows):
            def body(d, carry):
                col = jnp.zeros((_LANES,), jnp.int32) + d
                for b in range(_K):
                    for m in range(_CHUNK // _LANES):
                        v = plsc.load_gather(rows.at[b], [row_vecs[m], col])
                        tsp_v[b, d, pl.ds(m * _LANES, _LANES)] = v
                return carry
            lax.fori_loop(0, D, body, 0)

        def fire_writes(g, sem):
            for b in range(_K):
                s, boff = tile_sb(g * _K + b)
                pltpu.async_copy(
                    tsp_v.at[b],
                    out_hbm.at[s, :, pl.ds(col0 + boff, _CHUNK)], sem)

        def drain_writes(g, sem):
            for b in range(_K):
                s, boff = tile_sb(g * _K + b)
                pltpu.make_async_copy(
                    tsp_v.at[b],
                    out_hbm.at[s, :, pl.ds(col0 + boff, _CHUNK)], sem).wait()

        fire_gathers(0, rows_a, gs_a)
        fire_gathers(1, rows_b, gs_b)

        def body(t, carry):
            ga, gb = 2 * t, 2 * t + 1
            drain_gathers(ga, rows_a, gs_a)
            transpose_group(rows_a)
            fire_writes(ga, os_t)
            fire_gathers(ga + 2, rows_a, gs_a)
            drain_gathers(gb, rows_b, gs_b)
            drain_writes(ga, os_t)
            transpose_group(rows_b)
            fire_writes(gb, os_t)
            fire_gathers(gb + 2, rows_b, gs_b)
            drain_writes(gb, os_t)
            return carry

        lax.fori_loop(0, n_groups // 2 - 1, body, 0)

        ga, gb = n_groups - 2, n_groups - 1
        drain_gathers(ga, rows_a, gs_a)
        transpose_group(rows_a)
        fire_writes(ga, os_t)
        drain_gathers(gb, rows_b, gs_b)
        drain_writes(ga, os_t)
        transpose_group(rows_b)
        fire_writes(gb, os_t)
        drain_writes(gb, os_t)

    return gather_kernel


def kernel(token_ids, weight):
    B0, S = token_ids.shape
    V, D = weight.shape
    tids = token_ids.T.astype(jnp.int32)
    out_t = _make_gather(V, D, S, B0)(tids, weight)
    return jnp.transpose(out_t, (2, 0, 1))
